# Initial kernel scaffold; baseline (speedup 1.0000x reference)
import jax
import jax.numpy as jnp
from jax.experimental import pallas as pl


def kernel(nodevec1, nodevec2, nodevec3, node_embeddings):
    raise NotImplementedError("write your pallas kernel here")


# speedup vs baseline: 9.0842x; 9.0842x over previous
import functools

import jax
import jax.numpy as jnp
from jax import lax
from jax.experimental import pallas as pl
from jax.experimental.pallas import tpu as pltpu
from jax.experimental.pallas import tpu_sc as plsc

_KTOP = 8


def _topk_body(emb_ref, nv3_ref, idx_ref, flat_ref):
    m, n = emb_ref.shape[0], nv3_ref.shape[1]
    x = nv3_ref[0]
    e = emb_ref[...]
    logits = lax.dot_general(
        e, x, (((1,), (1,)), ((), ())), preferred_element_type=jnp.float32
    )
    col = lax.broadcasted_iota(jnp.int32, (m, n), 1)
    kcol = lax.broadcasted_iota(jnp.int32, (m, _KTOP), 1)
    idx_all = jnp.zeros((m, _KTOP), jnp.int32)
    cur = logits
    for k in range(_KTOP):
        mx = jnp.max(cur, axis=1, keepdims=True)
        idx = jnp.min(jnp.where(cur == mx, col, n), axis=1, keepdims=True)
        idx_all = jnp.where(kcol == k, idx, idx_all)
        cur = jnp.where(col == idx, -jnp.inf, cur)
    idx_ref[0] = idx_all
    flat_ref[0] = idx_all + pl.program_id(0) * n


def _topk_indices(emb, nv3):
    bt, n, e_dim = nv3.shape
    m = emb.shape[0]
    return pl.pallas_call(
        _topk_body,
        grid=(bt,),
        in_specs=[
            pl.BlockSpec((m, e_dim), lambda i: (0, 0)),
            pl.BlockSpec((1, n, e_dim), lambda i: (i, 0, 0)),
        ],
        out_specs=[
            pl.BlockSpec((1, m, _KTOP), lambda i: (i, 0, 0)),
            pl.BlockSpec((1, m, _KTOP), lambda i: (i, 0, 0)),
        ],
        out_shape=[
            jax.ShapeDtypeStruct((bt, m, _KTOP), jnp.int32),
            jax.ShapeDtypeStruct((bt, m, _KTOP), jnp.int32),
        ],
    )(emb, nv3)


def _sc_gather(t1, t2, idx3):
    nw, nch, ch = idx3.shape
    d = t1.shape[1]
    r_per = nch * ch
    info = plsc.get_sparse_core_info()
    nc = info.num_cores
    mesh = plsc.VectorSubcoreMesh(core_axis_name="c", subcore_axis_name="s")

    @functools.partial(
        pl.kernel,
        out_type=(
            jax.ShapeDtypeStruct((nw * r_per, d), jnp.float32),
            jax.ShapeDtypeStruct((nw * r_per, d), jnp.float32),
        ),
        mesh=mesh,
        scratch_types=[
            pltpu.VMEM((nch, ch), jnp.int32),
            pltpu.VMEM((ch, d), jnp.float32),
            pltpu.VMEM((ch, d), jnp.float32),
            pltpu.SemaphoreType.DMA,
            pltpu.SemaphoreType.DMA,
        ],
        compiler_params=pltpu.CompilerParams(use_tc_tiling_on_sc=False),
    )
    def gather_k(t1_hbm, t2_hbm, idx_hbm, out1_hbm, out2_hbm,
                 idx_v, buf1, buf2, sem1, sem2):
        wid = lax.axis_index("s") * nc + lax.axis_index("c")
        pltpu.sync_copy(idx_hbm.at[wid], idx_v)
        base = wid * r_per

        def body(j, carry):
            cp1 = pltpu.async_copy(t1_hbm.at[idx_v.at[j]], buf1, sem1)
            cp2 = pltpu.async_copy(t2_hbm.at[idx_v.at[j]], buf2, sem2)
            cp1.wait()
            cp2.wait()
            pltpu.sync_copy(buf1, out1_hbm.at[pl.ds(base + j * ch, ch)])
            pltpu.sync_copy(buf2, out2_hbm.at[pl.ds(base + j * ch, ch)])
            return carry

        lax.fori_loop(0, nch, body, 0)

    return gather_k(t1, t2, idx3)


def kernel(nodevec1, nodevec2, nodevec3, node_embeddings):
    b, t, n, d = nodevec1.shape
    m, e2 = node_embeddings.shape
    bt = b * t
    nv3 = nodevec3.reshape(bt, n, e2)
    idx, flat = _topk_indices(node_embeddings, nv3)
    indices = idx.reshape(b, t, m, _KTOP)

    info = plsc.get_sparse_core_info()
    nw = info.num_cores * info.num_subcores
    total = bt * m * _KTOP
    ch = 128
    nch = total // (nw * ch)
    idx3 = flat.reshape(nw, nch, ch)
    out1, out2 = _sc_gather(
        nodevec1.reshape(bt * n, d), nodevec2.reshape(bt * n, d), idx3
    )
    sel1 = out1.reshape(b, t, m, _KTOP, d)
    sel2 = out2.reshape(b, t, m, _KTOP, d)

    batch_indices = jnp.broadcast_to(
        jnp.arange(b, dtype=jnp.int32).reshape(b, 1, 1, 1), (b, t, m, _KTOP)
    )
    time_indices = jnp.broadcast_to(
        jnp.arange(t, dtype=jnp.int32).reshape(1, t, 1, 1), (b, t, m, _KTOP)
    )
    return sel1, sel2, batch_indices, time_indices, indices


# speedup vs baseline: 10.0327x; 1.1044x over previous
import functools

import jax
import jax.numpy as jnp
from jax import lax
from jax.experimental import pallas as pl
from jax.experimental.pallas import tpu as pltpu
from jax.experimental.pallas import tpu_sc as plsc

_KTOP = 8


def _topk_body(emb_ref, nv3_ref, idx_ref, flat_ref):
    m, n = emb_ref.shape[0], nv3_ref.shape[1]
    x = nv3_ref[0]
    e = emb_ref[...]
    logits = lax.dot_general(
        e, x, (((1,), (1,)), ((), ())), preferred_element_type=jnp.float32
    )
    colf = lax.broadcasted_iota(jnp.int32, (m, n), 1).astype(jnp.float32)
    kcol = lax.broadcasted_iota(jnp.int32, (m, _KTOP), 1)
    idxf_all = jnp.zeros((m, _KTOP), jnp.float32)
    cur = logits
    for k in range(_KTOP):
        mx = jnp.max(cur, axis=1, keepdims=True)
        idxf = jnp.min(
            jnp.where(cur == mx, colf, float(n)), axis=1, keepdims=True
        )
        idxf_all = jnp.where(kcol == k, idxf, idxf_all)
        cur = jnp.where(colf == idxf, -jnp.inf, cur)
    idx_all = idxf_all.astype(jnp.int32)
    idx_ref[0] = idx_all
    tiled = pltpu.repeat(idx_all, m, axis=1)
    liota = lax.broadcasted_iota(jnp.int32, (m, m * _KTOP), 1)
    riota = lax.broadcasted_iota(jnp.int32, (m, m * _KTOP), 0)
    picked = jnp.where(lax.shift_right_logical(liota, 3) == riota, tiled, 0)
    flat_row = jnp.sum(picked, axis=0, keepdims=True)
    flat_ref[0] = flat_row + pl.program_id(0) * n


def _topk_indices(emb, nv3):
    bt, n, e_dim = nv3.shape
    m = emb.shape[0]
    return pl.pallas_call(
        _topk_body,
        grid=(bt,),
        in_specs=[
            pl.BlockSpec((m, e_dim), lambda i: (0, 0)),
            pl.BlockSpec((1, n, e_dim), lambda i: (i, 0, 0)),
        ],
        out_specs=[
            pl.BlockSpec((1, m, _KTOP), lambda i: (i, 0, 0)),
            pl.BlockSpec((1, 1, m * _KTOP), lambda i: (i, 0, 0)),
        ],
        out_shape=[
            jax.ShapeDtypeStruct((bt, m, _KTOP), jnp.int32),
            jax.ShapeDtypeStruct((bt, 1, m * _KTOP), jnp.int32),
        ],
    )(emb, nv3)


def _sc_gather(t1, t2, idx3):
    nw, nch, ch = idx3.shape
    d = t1.shape[1]
    r_per = nch * ch
    info = plsc.get_sparse_core_info()
    nc = info.num_cores
    mesh = plsc.VectorSubcoreMesh(core_axis_name="c", subcore_axis_name="s")

    @functools.partial(
        pl.kernel,
        out_type=(
            jax.ShapeDtypeStruct((nw * r_per, d), jnp.float32),
            jax.ShapeDtypeStruct((nw * r_per, d), jnp.float32),
        ),
        mesh=mesh,
        scratch_types=[
            pltpu.VMEM((nch, ch), jnp.int32),
            pltpu.VMEM((ch, d), jnp.float32),
            pltpu.VMEM((ch, d), jnp.float32),
            pltpu.VMEM((ch, d), jnp.float32),
            pltpu.VMEM((ch, d), jnp.float32),
            pltpu.SemaphoreType.DMA,
            pltpu.SemaphoreType.DMA,
        ],
        compiler_params=pltpu.CompilerParams(use_tc_tiling_on_sc=False),
    )
    def gather_k(t1_hbm, t2_hbm, idx_hbm, out1_hbm, out2_hbm,
                 idx_v, buf1a, buf2a, buf1b, buf2b, sema, semb):
        wid = lax.axis_index("s") * nc + lax.axis_index("c")
        pltpu.sync_copy(idx_hbm.at[wid], idx_v)
        base = wid * r_per

        def body(j2, carry):
            ja = 2 * j2
            jb = 2 * j2 + 1
            cp1a = pltpu.async_copy(t1_hbm.at[idx_v.at[ja]], buf1a, sema)
            cp2a = pltpu.async_copy(t2_hbm.at[idx_v.at[ja]], buf2a, sema)
            cp1b = pltpu.async_copy(t1_hbm.at[idx_v.at[jb]], buf1b, semb)
            cp2b = pltpu.async_copy(t2_hbm.at[idx_v.at[jb]], buf2b, semb)
            cp1a.wait()
            cp2a.wait()
            pltpu.sync_copy(buf1a, out1_hbm.at[pl.ds(base + ja * ch, ch)])
            pltpu.sync_copy(buf2a, out2_hbm.at[pl.ds(base + ja * ch, ch)])
            cp1b.wait()
            cp2b.wait()
            pltpu.sync_copy(buf1b, out1_hbm.at[pl.ds(base + jb * ch, ch)])
            pltpu.sync_copy(buf2b, out2_hbm.at[pl.ds(base + jb * ch, ch)])
            return carry

        lax.fori_loop(0, nch // 2, body, 0)

    return gather_k(t1, t2, idx3)


def kernel(nodevec1, nodevec2, nodevec3, node_embeddings):
    b, t, n, d = nodevec1.shape
    m, e2 = node_embeddings.shape
    bt = b * t
    nv3 = nodevec3.reshape(bt, n, e2)
    idx, flat = _topk_indices(node_embeddings, nv3)
    indices = idx.reshape(b, t, m, _KTOP)

    info = plsc.get_sparse_core_info()
    nw = info.num_cores * info.num_subcores
    total = bt * m * _KTOP
    ch = 128
    nch = total // (nw * ch)
    idx3 = flat.reshape(nw, nch, ch)
    out1, out2 = _sc_gather(
        nodevec1.reshape(bt * n, d), nodevec2.reshape(bt * n, d), idx3
    )
    sel1 = out1.reshape(b, t, m, _KTOP, d)
    sel2 = out2.reshape(b, t, m, _KTOP, d)

    batch_indices = jnp.broadcast_to(
        jnp.arange(b, dtype=jnp.int32).reshape(b, 1, 1, 1), (b, t, m, _KTOP)
    )
    time_indices = jnp.broadcast_to(
        jnp.arange(t, dtype=jnp.int32).reshape(1, t, 1, 1), (b, t, m, _KTOP)
    )
    return sel1, sel2, batch_indices, time_indices, indices
